# baseline (speedup 1.0000x reference)
import jax
import jax.numpy as jnp
from jax import lax
from jax.experimental import pallas as pl
from jax.experimental.pallas import tpu as pltpu

B = 4
T = 2048
V = 1000
NSPLIT = 8
ROWS = T // NSPLIT
DCH = 256
NDCH = ROWS // DCH


def _loss_kernel(*refs):
    lp_refs = refs[:NSPLIT]
    tx_ref, num_ref, den_ref, acc_ref = refs[NSPLIT:]
    i = pl.program_id(0)

    @pl.when(i == 0)
    def _():
        acc_ref[...] = jnp.zeros((T, 128), jnp.float32)
        ap = lax.broadcasted_iota(jnp.int32, (B, T), 1)
        eb = jnp.min(jnp.where(tx_ref[...] == 0, ap, T), axis=1,
                     keepdims=True)
        den = jnp.sum(jnp.minimum(eb + 1, T).astype(jnp.float32),
                      keepdims=True)
        den_ref[...] = den.reshape(1, 1)

    row = tx_ref[pl.ds(i, 1), :]
    tpos = lax.broadcasted_iota(jnp.int32, (1, T), 1)
    e = jnp.min(jnp.where(row == 0, tpos, T))

    si = lax.broadcasted_iota(jnp.int32, (DCH, DCH), 0)
    li = lax.broadcasted_iota(jnp.int32, (DCH, DCH), 1)
    diag = si == li
    vpos = lax.broadcasted_iota(jnp.int32, (ROWS, V), 1)

    for q in range(NSPLIT):
        t0 = q * ROWS
        parts = []
        for k in range(NDCH):
            ids = tx_ref[pl.ds(i, 1), pl.ds(t0 + k * DCH, DCH)]
            rb = jnp.broadcast_to(ids, (DCH, DCH))
            parts.append(jnp.sum(jnp.where(diag, rb, 0), axis=1,
                                 keepdims=True))
        cols = parts[0] if NDCH == 1 else jnp.concatenate(parts, axis=0)

        tvec = t0 + lax.broadcasted_iota(jnp.int32, (ROWS, 1), 0)
        cm = jnp.where(tvec <= e, cols, -1)

        lp = lp_refs[q][0]
        sel = jnp.where(vpos == cm, lp, 0.0)
        part = sel[:, 0:128]
        for s in range(1, 7):
            part = part + sel[:, s * 128:(s + 1) * 128]
        tail = jnp.concatenate(
            [sel[:, 896:1000], jnp.zeros((ROWS, 24), jnp.float32)], axis=1)
        acc_ref[pl.ds(t0, ROWS), :] += part + tail

    @pl.when(i == B - 1)
    def _():
        num_ref[...] = jnp.sum(acc_ref[...], keepdims=True).reshape(1, 1)


def _make_spec(q):
    return pl.BlockSpec((1, ROWS, V), lambda i, _q=q: (i, _q, 0))


@jax.jit
def kernel(log_probs, text_encoded):
    tx = text_encoded.astype(jnp.int32)

    num, den = pl.pallas_call(
        _loss_kernel,
        grid=(B,),
        in_specs=[_make_spec(q) for q in range(NSPLIT)] + [
            pl.BlockSpec((B, T), lambda i: (0, 0)),
        ],
        out_specs=[
            pl.BlockSpec((1, 1), lambda i: (0, 0)),
            pl.BlockSpec((1, 1), lambda i: (0, 0)),
        ],
        out_shape=[
            jax.ShapeDtypeStruct((1, 1), jnp.float32),
            jax.ShapeDtypeStruct((1, 1), jnp.float32),
        ],
        scratch_shapes=[pltpu.VMEM((T, 128), jnp.float32)],
        compiler_params=pltpu.CompilerParams(
            dimension_semantics=("arbitrary",),
        ),
    )(*([log_probs] * NSPLIT + [tx]))

    return -num[0, 0] / den[0, 0]
